# Initial kernel scaffold; baseline (speedup 1.0000x reference)
import jax
import jax.numpy as jnp
from jax.experimental import pallas as pl


def kernel(station_ids, id_emb, t_from_A, W1, b1, W2, b2):
    raise NotImplementedError("write your pallas kernel here")


# speedup vs baseline: 14.2672x; 14.2672x over previous
import functools

import jax
import jax.numpy as jnp
from jax import lax
from jax.experimental import pallas as pl
from jax.experimental.pallas import tpu as pltpu
from jax.experimental.pallas import tpu_sc as plsc

_D_ID = 32
_D_T = 32
_D_OUT = _D_ID + _D_T


def _table_body(id_ref, t_ref, w1_ref, b1_ref, w2t_ref, b2_ref, out_ref):
    t = t_ref[...]
    t_scale = jnp.max(t) + 1e-6
    ta = t / t_scale
    h = jnp.maximum(ta * w1_ref[...] + b1_ref[...], 0.0)
    e_t = jnp.dot(h, w2t_ref[...], preferred_element_type=jnp.float32)
    e_t = e_t + b2_ref[...]
    out_ref[:, :_D_ID] = id_ref[...]
    out_ref[:, _D_ID:] = e_t


def _build_table(id_emb, t_from_A, W1, b1, W2, b2):
    n = id_emb.shape[0]
    return pl.pallas_call(
        _table_body,
        out_shape=jax.ShapeDtypeStruct((n, _D_OUT), jnp.float32),
    )(
        id_emb,
        t_from_A.reshape(n, 1),
        W1.reshape(1, _D_T),
        b1.reshape(1, _D_T),
        W2.T,
        b2.reshape(1, _D_T),
    )


_NC = 2
_NS = 16
_NW = _NC * _NS
_CHUNK = 512


def _gather_body(n_chunks, table_hbm, idx_hbm, out_hbm, idx_v, rows_v, sem):
    wid = lax.axis_index("s") * _NC + lax.axis_index("c")
    base = wid * (n_chunks * _CHUNK)

    @pl.loop(0, n_chunks)
    def _chunk(c):
        start = base + c * _CHUNK
        pltpu.sync_copy(idx_hbm.at[pl.ds(start, _CHUNK)], idx_v)
        pltpu.async_copy(table_hbm.at[idx_v], rows_v, sem).wait()
        pltpu.sync_copy(rows_v, out_hbm.at[pl.ds(start, _CHUNK)])


def _gather(table, idx_flat):
    n_idx = idx_flat.shape[0]
    assert n_idx % (_NW * _CHUNK) == 0
    n_chunks = n_idx // (_NW * _CHUNK)
    mesh = plsc.VectorSubcoreMesh(core_axis_name="c", subcore_axis_name="s")
    k = pl.kernel(
        functools.partial(_gather_body, n_chunks),
        out_type=jax.ShapeDtypeStruct((n_idx, _D_OUT), jnp.float32),
        mesh=mesh,
        scratch_types=[
            pltpu.VMEM((_CHUNK,), jnp.int32),
            pltpu.VMEM((_CHUNK, _D_OUT), jnp.float32),
            pltpu.SemaphoreType.DMA,
        ],
        compiler_params=pltpu.CompilerParams(use_tc_tiling_on_sc=False),
    )
    return k(table, idx_flat)


@jax.jit
def kernel(station_ids, id_emb, t_from_A, W1, b1, W2, b2):
    B, L = station_ids.shape
    table = _build_table(id_emb, t_from_A, W1, b1, W2, b2)
    out = _gather(table, station_ids.reshape(B * L))
    return out.reshape(B, L, _D_OUT)
